# Initial kernel scaffold; baseline (speedup 1.0000x reference)
import jax
import jax.numpy as jnp
from jax.experimental import pallas as pl


def kernel(x, Wg, W1, W2):
    raise NotImplementedError("write your pallas kernel here")


# speedup vs baseline: 6.2284x; 6.2284x over previous
import functools

import jax
import jax.numpy as jnp
from jax.experimental import pallas as pl
from jax.experimental.pallas import tpu as pltpu

TOPK = 2
NEXP = 8


def _moe_body(x_ref, wg_ref, w1_ref, w2_ref,
              out_ref, lb_ref, rz_ref, cnt_ref, w_scr):
    e = pl.program_id(0)

    @pl.when(e == 0)
    def _router():
        xf = x_ref[...]
        logits = jnp.dot(xf, wg_ref[...], preferred_element_type=jnp.float32)
        m = jnp.max(logits, axis=-1, keepdims=True)
        ex = jnp.exp(logits - m)
        ssum = jnp.sum(ex, axis=-1, keepdims=True)
        scores = ex / ssum
        rz = jnp.log(ssum) + m
        rz_ref[0, 0] = jnp.mean(rz * rz)

        col = jax.lax.broadcasted_iota(jnp.int32, scores.shape, 1)
        m1 = jnp.max(scores, axis=-1, keepdims=True)
        idx1 = jnp.min(jnp.where(scores == m1, col, NEXP), axis=-1,
                       keepdims=True)
        sel1 = col == idx1
        s_masked = jnp.where(sel1, -jnp.inf, scores)
        m2 = jnp.max(s_masked, axis=-1, keepdims=True)
        idx2 = jnp.min(jnp.where(s_masked == m2, col, NEXP), axis=-1,
                       keepdims=True)
        sel2 = col == idx2
        picked = sel1 | sel2
        w = jnp.where(picked, scores, 0.0)
        w_scr[...] = w

        counts = jnp.sum(picked.astype(jnp.int32), axis=0)
        cnt_ref[...] = counts[None, :]
        seg_sum = jnp.sum(w, axis=0)
        total = jnp.float32(w.shape[0] * TOPK)
        dist = counts.astype(jnp.float32) / total
        avg = seg_sum / jnp.maximum(counts.astype(jnp.float32), 1.0)
        lb_ref[0, 0] = jnp.sum(dist * avg) * NEXP

    wall = w_scr[...]
    ecol = jax.lax.broadcasted_iota(jnp.int32, wall.shape, 1)
    we = jnp.sum(jnp.where(ecol == e, wall, 0.0), axis=1, keepdims=True)
    xin = x_ref[...] * we
    h = jnp.maximum(
        jnp.dot(xin, w1_ref[0], preferred_element_type=jnp.float32), 0.0)
    contrib = jnp.dot(h, w2_ref[0], preferred_element_type=jnp.float32)

    @pl.when(e == 0)
    def _init():
        out_ref[...] = contrib

    @pl.when(e > 0)
    def _acc():
        out_ref[...] += contrib


@functools.partial(jax.jit, static_argnames=())
def kernel(x, Wg, W1, W2):
    B, S, D = x.shape
    E = W1.shape[0]
    T = B * S
    xf = x.reshape(T, D)

    out, lb, rz, cnt = pl.pallas_call(
        _moe_body,
        grid=(E,),
        in_specs=[
            pl.BlockSpec((T, D), lambda e: (0, 0)),
            pl.BlockSpec((D, E), lambda e: (0, 0)),
            pl.BlockSpec((1, D, W1.shape[2]), lambda e: (e, 0, 0)),
            pl.BlockSpec((1, W2.shape[1], D), lambda e: (e, 0, 0)),
        ],
        out_specs=[
            pl.BlockSpec((T, D), lambda e: (0, 0)),
            pl.BlockSpec(memory_space=pltpu.SMEM),
            pl.BlockSpec(memory_space=pltpu.SMEM),
            pl.BlockSpec((1, E), lambda e: (0, 0)),
        ],
        out_shape=[
            jax.ShapeDtypeStruct((T, D), jnp.float32),
            jax.ShapeDtypeStruct((1, 1), jnp.float32),
            jax.ShapeDtypeStruct((1, 1), jnp.float32),
            jax.ShapeDtypeStruct((1, E), jnp.int32),
        ],
        scratch_shapes=[pltpu.VMEM((T, E), jnp.float32)],
    )(xf, Wg, W1, W2)

    return (out.reshape(B, S, D), lb.reshape(()), rz.reshape(()),
            cnt.reshape(E))
